# baseline (speedup 1.0000x reference)
import functools

import jax
import jax.numpy as jnp
from jax import lax
from jax.experimental import pallas as pl
from jax.experimental.pallas import tpu as pltpu
from jax.experimental.pallas import tpu_sc as plsc

N = 16384
A = 128
C = 100
C_PAD = 112
NC = 2
NS = 16
NW = NC * NS
RPW = N // NW
LANES = 16
CHUNK = 128
NCHUNK = RPW // CHUNK
NBUF = 3
ZROWS = C_PAD // NS


@functools.partial(
    pl.kernel,
    out_type=(
        jax.ShapeDtypeStruct((NC, C_PAD, A), jnp.float32),
        jax.ShapeDtypeStruct((NC, C_PAD, A), jnp.float32),
        jax.ShapeDtypeStruct((NC, C_PAD, A), jnp.float32),
    ),
    mesh=plsc.VectorSubcoreMesh(core_axis_name="c", subcore_axis_name="s"),
    scratch_types=(
        pltpu.VMEM((NBUF, CHUNK, A), jnp.float32),
        pltpu.VMEM((NCHUNK, CHUNK), jnp.int32),
        pltpu.VMEM((C_PAD, A), jnp.float32),
        pltpu.VMEM((C_PAD, A), jnp.float32),
        pltpu.VMEM((1, C_PAD), jnp.int32),
        pltpu.VMEM((ZROWS, A), jnp.float32),
        pltpu.VMEM_SHARED((C_PAD, A), jnp.float32),
        pltpu.VMEM_SHARED((C_PAD, A), jnp.float32),
        pltpu.VMEM_SHARED((C_PAD, A), jnp.float32),
        pltpu.SemaphoreType.DMA,
        pltpu.SemaphoreType.DMA,
        pltpu.SemaphoreType.DMA,
    ),
)
def _segment_sums(feat_hbm, lbl_hbm, psum_hbm, psq_hbm, pcnt_hbm,
                  feat_v, lbl_v, asq_v, acnt_v, iota_v, zero_v,
                  ssum, ssq, scnt, fsem, lsem, sem0):
    cid = lax.axis_index("c")
    sid = lax.axis_index("s")
    wid = sid * NC + cid
    base = wid * RPW

    pltpu.async_copy(feat_hbm.at[pl.ds(base, CHUNK)], feat_v.at[0], fsem)
    lbl_dma = pltpu.async_copy(lbl_hbm.at[wid], lbl_v, lsem)

    zero16 = jnp.zeros((LANES,), jnp.float32)

    def zrow(r, carry):
        for ch in range(A // LANES):
            zero_v[r, pl.ds(ch * LANES, LANES)] = zero16
        return carry

    lax.fori_loop(0, ZROWS, zrow, 0)
    row0 = sid * ZROWS
    pltpu.sync_copy(zero_v, ssum.at[pl.ds(row0, ZROWS)])
    pltpu.sync_copy(zero_v, ssq.at[pl.ds(row0, ZROWS)])
    pltpu.sync_copy(zero_v, scnt.at[pl.ds(row0, ZROWS)])

    def azrow(r, carry):
        for ch in range(A // LANES):
            asq_v[r, pl.ds(ch * LANES, LANES)] = zero16
        acnt_v[r, pl.ds(0, LANES)] = zero16
        return carry

    lax.fori_loop(0, C_PAD, azrow, 0)

    for g in range(C_PAD // LANES):
        iota_v[0, pl.ds(g * LANES, LANES)] = (
            lax.iota(jnp.int32, LANES) + g * LANES)

    plsc.subcore_barrier()

    lbl_dma.wait()

    one16 = jnp.ones((LANES,), jnp.float32)

    def cntgroup(g, carry):
        lbls = lbl_v[g // (CHUNK // LANES),
                     pl.ds((g % (CHUNK // LANES)) * LANES, LANES)]
        for r in range(LANES):
            plsc.addupdate(acnt_v.at[lbls[r], pl.ds(0, LANES)], one16)
        return carry

    lax.fori_loop(0, RPW // LANES, cntgroup, 0)

    sum_scatters = [None] * NCHUNK

    for j in range(NCHUNK):
        buf = j % NBUF
        pltpu.make_async_copy(
            feat_hbm.at[pl.ds(base + j * CHUNK, CHUNK)], feat_v.at[buf], fsem
        ).wait()

        sum_scatters[j] = pltpu.async_copy(
            feat_v.at[buf], ssum.at[lbl_v.at[j]], sem0, add=True)

        if j + 1 < NCHUNK:
            nslot = (j + 1) % NBUF
            if j + 1 >= NBUF:
                sum_scatters[j + 1 - NBUF].wait()
                sum_scatters[j + 1 - NBUF] = None
            pltpu.async_copy(
                feat_hbm.at[pl.ds(base + (j + 1) * CHUNK, CHUNK)],
                feat_v.at[nslot], fsem)

        def sqgroup(g, carry):
            i0 = g * LANES
            lbls = lbl_v[j, pl.ds(i0, LANES)]
            xs = [feat_v[buf, i0, pl.ds(ch * LANES, LANES)]
                  for ch in range(A // LANES)]
            for r in range(LANES):
                l = lbls[r]
                if r + 1 < LANES:
                    nxt = [feat_v[buf, i0 + r + 1, pl.ds(ch * LANES, LANES)]
                           for ch in range(A // LANES)]
                for ch in range(A // LANES):
                    plsc.addupdate(asq_v.at[l, pl.ds(ch * LANES, LANES)],
                                   xs[ch] * xs[ch])
                if r + 1 < LANES:
                    xs = nxt
            return carry

        lax.fori_loop(0, CHUNK // LANES, sqgroup, 0)

    for pending in sum_scatters:
        if pending is not None:
            pending.wait()

    pltpu.sync_copy(asq_v, ssq.at[iota_v.at[0]], add=True)
    pltpu.sync_copy(acnt_v, scnt.at[iota_v.at[0]], add=True)

    plsc.subcore_barrier()

    @pl.when(sid == 0)
    def _():
        pltpu.sync_copy(ssum, psum_hbm.at[cid])
        pltpu.sync_copy(ssq, psq_hbm.at[cid])
        pltpu.sync_copy(scnt, pcnt_hbm.at[cid])


def _combine(psum_ref, psq_ref, pcnt_ref, count_ref, mean_ref, cov_ref,
             cov_out, mean_out, cnt_out):
    sum_x = psum_ref[0] + psum_ref[1]
    sum_x2 = psq_ref[0] + psq_ref[1]
    counts_f = (pcnt_ref[0] + pcnt_ref[1])[:, 0:1]
    count = count_ref[...]
    mean = mean_ref[...]
    cov = cov_ref[...]

    amount = jnp.where(counts_f == 0.0, 1.0, counts_f)
    ave = sum_x / amount
    var_temp = (sum_x2 - 2.0 * ave * sum_x + counts_f * ave * ave) / amount
    denom = counts_f + count
    w = jnp.where(denom == 0.0, 0.0,
                  counts_f / jnp.where(denom == 0.0, 1.0, denom))
    cov_out[...] = cov * (1.0 - w) + var_temp * w + w * (1.0 - w) * (mean - ave) ** 2
    mean_out[...] = mean * (1.0 - w) + ave * w
    cnt_out[...] = count + counts_f


def kernel(features, labels, count, mean, cov):
    psum, psq, pcnt = _segment_sums(
        features, labels.reshape(NW, NCHUNK, CHUNK))
    pad = ((0, C_PAD - C), (0, 0))
    cov_new, mean_new, cnt_new = pl.pallas_call(
        _combine,
        out_shape=(
            jax.ShapeDtypeStruct((C_PAD, A), jnp.float32),
            jax.ShapeDtypeStruct((C_PAD, A), jnp.float32),
            jax.ShapeDtypeStruct((C_PAD, 1), jnp.float32),
        ),
    )(psum, psq, pcnt,
      jnp.pad(count[:, None], pad), jnp.pad(mean, pad), jnp.pad(cov, pad))
    return cov_new[:C], mean_new[:C], cnt_new[:C, 0]


# speedup vs baseline: 1.0835x; 1.0835x over previous
import functools

import jax
import jax.numpy as jnp
from jax import lax
from jax.experimental import pallas as pl
from jax.experimental.pallas import tpu as pltpu
from jax.experimental.pallas import tpu_sc as plsc

N = 16384
A = 128
C = 100
C_PAD = 112
NC = 2
NS = 16
NW = NC * NS
RPW = N // NW
LANES = 16
CHUNK = 128
NCHUNK = RPW // CHUNK
NBUF = 3
ZROWS = C_PAD // NS


@functools.partial(
    pl.kernel,
    out_type=(
        jax.ShapeDtypeStruct((NC, C_PAD, A), jnp.float32),
        jax.ShapeDtypeStruct((NC, C_PAD, A), jnp.float32),
        jax.ShapeDtypeStruct((NC, C_PAD, A), jnp.float32),
    ),
    mesh=plsc.VectorSubcoreMesh(core_axis_name="c", subcore_axis_name="s"),
    scratch_types=(
        pltpu.VMEM((NBUF, CHUNK, A), jnp.float32),
        pltpu.VMEM((NCHUNK, CHUNK), jnp.int32),
        pltpu.VMEM((C_PAD, A), jnp.float32),
        pltpu.VMEM((C_PAD, A), jnp.float32),
        pltpu.VMEM((1, C_PAD), jnp.int32),
        pltpu.VMEM((ZROWS, A), jnp.float32),
        pltpu.VMEM_SHARED((C_PAD, A), jnp.float32),
        pltpu.VMEM_SHARED((C_PAD, A), jnp.float32),
        pltpu.VMEM_SHARED((C_PAD, A), jnp.float32),
        pltpu.SemaphoreType.DMA,
        pltpu.SemaphoreType.DMA,
        pltpu.SemaphoreType.DMA,
    ),
)
def _segment_sums(feat_hbm, lbl_hbm, psum_hbm, psq_hbm, pcnt_hbm,
                  feat_v, lbl_v, asq_v, acnt_v, iota_v, zero_v,
                  ssum, ssq, scnt, fsem, lsem, sem0):
    cid = lax.axis_index("c")
    sid = lax.axis_index("s")
    wid = sid * NC + cid
    base = wid * RPW

    pltpu.async_copy(feat_hbm.at[pl.ds(base, CHUNK)], feat_v.at[0], fsem)
    lbl_dma = pltpu.async_copy(lbl_hbm.at[wid], lbl_v, lsem)

    zero16 = jnp.zeros((LANES,), jnp.float32)

    def zrow(r, carry):
        for ch in range(A // LANES):
            zero_v[r, pl.ds(ch * LANES, LANES)] = zero16
        return carry

    lax.fori_loop(0, ZROWS, zrow, 0)
    row0 = sid * ZROWS
    pltpu.sync_copy(zero_v, ssum.at[pl.ds(row0, ZROWS)])
    pltpu.sync_copy(zero_v, ssq.at[pl.ds(row0, ZROWS)])
    pltpu.sync_copy(zero_v, scnt.at[pl.ds(row0, ZROWS)])

    def azrow(r, carry):
        for ch in range(A // LANES):
            asq_v[r, pl.ds(ch * LANES, LANES)] = zero16
        acnt_v[r, pl.ds(0, LANES)] = zero16
        return carry

    lax.fori_loop(0, C_PAD, azrow, 0)

    for g in range(C_PAD // LANES):
        iota_v[0, pl.ds(g * LANES, LANES)] = (
            lax.iota(jnp.int32, LANES) + g * LANES)

    plsc.subcore_barrier()

    lbl_dma.wait()

    one16 = jnp.ones((LANES,), jnp.float32)

    def cntgroup(g, carry):
        lbls = lbl_v[g // (CHUNK // LANES),
                     pl.ds((g % (CHUNK // LANES)) * LANES, LANES)]
        for r in range(LANES):
            plsc.addupdate(acnt_v.at[lbls[r], pl.ds(0, LANES)], one16)
        return carry

    lax.fori_loop(0, RPW // LANES, cntgroup, 0)

    sum_scatters = [None] * NCHUNK

    for j in range(NCHUNK):
        buf = j % NBUF
        pltpu.make_async_copy(
            feat_hbm.at[pl.ds(base + j * CHUNK, CHUNK)], feat_v.at[buf], fsem
        ).wait()

        sum_scatters[j] = pltpu.async_copy(
            feat_v.at[buf], ssum.at[lbl_v.at[j]], sem0, add=True)

        if j + 1 < NCHUNK:
            nslot = (j + 1) % NBUF
            if j + 1 >= NBUF:
                sum_scatters[j + 1 - NBUF].wait()
                sum_scatters[j + 1 - NBUF] = None
            pltpu.async_copy(
                feat_hbm.at[pl.ds(base + (j + 1) * CHUNK, CHUNK)],
                feat_v.at[nslot], fsem)

        def sqgroup(g, carry):
            i0 = g * LANES
            lbls = lbl_v[j, pl.ds(i0, LANES)]
            xs = [feat_v[buf, i0, pl.ds(ch * LANES, LANES)]
                  for ch in range(A // LANES)]
            for r in range(LANES):
                l = lbls[r]
                if r + 1 < LANES:
                    nxt = [feat_v[buf, i0 + r + 1, pl.ds(ch * LANES, LANES)]
                           for ch in range(A // LANES)]
                for ch in range(A // LANES):
                    plsc.addupdate(asq_v.at[l, pl.ds(ch * LANES, LANES)],
                                   xs[ch] * xs[ch])
                if r + 1 < LANES:
                    xs = nxt
            return carry

        lax.fori_loop(0, CHUNK // LANES, sqgroup, 0)

    for pending in sum_scatters:
        if pending is not None:
            pending.wait()

    pltpu.sync_copy(asq_v, ssq.at[iota_v.at[0]], add=True)
    pltpu.sync_copy(acnt_v, scnt.at[iota_v.at[0]], add=True)

    plsc.subcore_barrier()

    @pl.when(sid == 0)
    def _():
        pltpu.sync_copy(ssum, psum_hbm.at[cid])
        pltpu.sync_copy(ssq, psq_hbm.at[cid])
        pltpu.sync_copy(scnt, pcnt_hbm.at[cid])


def _combine(psum_ref, psq_ref, pcnt_ref, count_ref, mean_ref, cov_ref,
             cov_out, mean_out, cnt_out):
    sum_x = (psum_ref[0] + psum_ref[1])[:C]
    sum_x2 = (psq_ref[0] + psq_ref[1])[:C]
    counts_f = (pcnt_ref[0] + pcnt_ref[1])[:C, 0:1]
    count = count_ref[...]
    mean = mean_ref[...]
    cov = cov_ref[...]

    amount = jnp.where(counts_f == 0.0, 1.0, counts_f)
    ave = sum_x / amount
    var_temp = (sum_x2 - 2.0 * ave * sum_x + counts_f * ave * ave) / amount
    denom = counts_f + count
    w = jnp.where(denom == 0.0, 0.0,
                  counts_f / jnp.where(denom == 0.0, 1.0, denom))
    cov_out[...] = cov * (1.0 - w) + var_temp * w + w * (1.0 - w) * (mean - ave) ** 2
    mean_out[...] = mean * (1.0 - w) + ave * w
    cnt_out[...] = count + counts_f


def kernel(features, labels, count, mean, cov):
    psum, psq, pcnt = _segment_sums(
        features, labels.reshape(NW, NCHUNK, CHUNK))
    cov_new, mean_new, cnt_new = pl.pallas_call(
        _combine,
        out_shape=(
            jax.ShapeDtypeStruct((C, A), jnp.float32),
            jax.ShapeDtypeStruct((C, A), jnp.float32),
            jax.ShapeDtypeStruct((C, 1), jnp.float32),
        ),
    )(psum, psq, pcnt, count[:, None], mean, cov)
    return cov_new, mean_new, cnt_new[:, 0]
